# Initial kernel scaffold; baseline (speedup 1.0000x reference)
import functools

import jax
import jax.numpy as jnp
from jax import lax
from jax.experimental import pallas as pl
from jax.experimental.pallas import tpu as pltpu
from jax.experimental.pallas import tpu_sc as plsc

N = 10000
E = 320000
D = 128
C = 64

NC = 2
NS = 16
NW = NC * NS
EPT = E // NW
EB = 80
NBLK = EPT // EB
RPT = N // NS

_mesh = plsc.VectorSubcoreMesh(core_axis_name="c", subcore_axis_name="s")


@functools.partial(
    pl.kernel,
    out_type=jax.ShapeDtypeStruct((NC * N, 16), jnp.float32),
    mesh=_mesh,
    scratch_types=[
        pltpu.VMEM((1, EB), jnp.int32),
        pltpu.VMEM((EB, 16), jnp.float32),
        pltpu.VMEM_SHARED((N, 16), jnp.float32),
    ],
)
def _sc_degree(col_hbm, ones_hbm, zero_hbm, out_hbm, idx_c, ones_v, accum):
    cid = lax.axis_index("c")
    sid = lax.axis_index("s")
    wid = cid * NS + sid
    r0 = sid * RPT
    pltpu.sync_copy(zero_hbm, accum.at[pl.ds(r0, RPT)])
    pltpu.sync_copy(ones_hbm, ones_v)
    plsc.subcore_barrier()
    ebase = wid * EPT

    @pl.loop(0, NBLK)
    def _(j):
        pltpu.sync_copy(col_hbm.at[pl.ds(ebase + j * EB, EB)], idx_c.at[0])
        pltpu.sync_copy(ones_v, accum.at[idx_c.at[0]], add=True)

    plsc.subcore_barrier()
    pltpu.sync_copy(accum.at[pl.ds(r0, RPT)],
                    out_hbm.at[pl.ds(cid * N + r0, RPT)])


@functools.partial(
    pl.kernel,
    out_type=jax.ShapeDtypeStruct((NC * N, C), jnp.float32),
    mesh=_mesh,
    scratch_types=[
        pltpu.VMEM((1, EB), jnp.int32),
        pltpu.VMEM((1, EB), jnp.int32),
        pltpu.VMEM((EB, C), jnp.float32),
        pltpu.VMEM_SHARED((N, C), jnp.float32),
    ],
)
def _sc_hop(g_hbm, row_hbm, col_hbm, zero_hbm, out_hbm,
            idx_r, idx_c, rows_v, accum):
    cid = lax.axis_index("c")
    sid = lax.axis_index("s")
    wid = cid * NS + sid
    r0 = sid * RPT

    @pl.when(cid == 0)
    def _():
        pltpu.sync_copy(g_hbm.at[pl.ds(r0, RPT)], accum.at[pl.ds(r0, RPT)])

    @pl.when(cid != 0)
    def _():
        pltpu.sync_copy(zero_hbm, accum.at[pl.ds(r0, RPT)])

    plsc.subcore_barrier()
    ebase = wid * EPT

    @pl.loop(0, NBLK)
    def _(j):
        base = ebase + j * EB
        pltpu.sync_copy(row_hbm.at[pl.ds(base, EB)], idx_r.at[0])
        pltpu.sync_copy(col_hbm.at[pl.ds(base, EB)], idx_c.at[0])
        pltpu.sync_copy(g_hbm.at[idx_r.at[0]], rows_v)
        pltpu.sync_copy(rows_v, accum.at[idx_c.at[0]], add=True)

    plsc.subcore_barrier()
    pltpu.sync_copy(accum.at[pl.ds(r0, RPT)],
                    out_hbm.at[pl.ds(cid * N + r0, RPT)])


RB = 1000


def _z_body(x_ref, w_ref, z_ref):
    z_ref[...] = lax.dot_general(x_ref[...], w_ref[...],
                                 (((1,), (1,)), ((), ())),
                                 preferred_element_type=jnp.float32)


def _tc_matmul(x, W):
    return pl.pallas_call(
        _z_body,
        grid=(N // RB,),
        in_specs=[pl.BlockSpec((RB, D), lambda i: (i, 0)),
                  pl.BlockSpec((C, D), lambda i: (0, 0))],
        out_specs=pl.BlockSpec((RB, C), lambda i: (i, 0)),
        out_shape=jax.ShapeDtypeStruct((N, C), jnp.float32),
    )(x, W)


def _prep_body(degp_ref, z_ref, g0_ref, dis_ref):
    d = degp_ref[...]
    deg = d[0, :, 0] + d[1, :, 0] + 1.0
    dis = lax.rsqrt(deg)
    g0_ref[...] = z_ref[...] * dis[:, None]
    dis_ref[...] = dis[:, None]


def _tc_prep(degp, z):
    return pl.pallas_call(
        _prep_body,
        grid=(N // RB,),
        in_specs=[pl.BlockSpec((2, RB, 16), lambda i: (0, i, 0)),
                  pl.BlockSpec((RB, C), lambda i: (i, 0))],
        out_specs=[pl.BlockSpec((RB, C), lambda i: (i, 0)),
                   pl.BlockSpec((RB, 1), lambda i: (i, 0))],
        out_shape=[jax.ShapeDtypeStruct((N, C), jnp.float32),
                   jax.ShapeDtypeStruct((N, 1), jnp.float32)],
    )(degp, z)


def _mid_body(p_ref, dis_ref, o_ref):
    p = p_ref[...]
    dis = dis_ref[...]
    o_ref[...] = (p[0] + p[1]) * (dis * dis)


def _tc_mid(p, dis):
    return pl.pallas_call(
        _mid_body,
        grid=(N // RB,),
        in_specs=[pl.BlockSpec((2, RB, C), lambda i: (0, i, 0)),
                  pl.BlockSpec((RB, 1), lambda i: (i, 0))],
        out_specs=pl.BlockSpec((RB, C), lambda i: (i, 0)),
        out_shape=jax.ShapeDtypeStruct((N, C), jnp.float32),
    )(p, dis)


def _fin_body(p_ref, dis_ref, b_ref, o_ref):
    p = p_ref[...]
    o_ref[...] = (p[0] + p[1]) * dis_ref[...] + b_ref[...]


def _tc_fin(p, dis, b2):
    return pl.pallas_call(
        _fin_body,
        grid=(N // RB,),
        in_specs=[pl.BlockSpec((2, RB, C), lambda i: (0, i, 0)),
                  pl.BlockSpec((RB, 1), lambda i: (i, 0)),
                  pl.BlockSpec((1, C), lambda i: (0, 0))],
        out_specs=pl.BlockSpec((RB, C), lambda i: (i, 0)),
        out_shape=jax.ShapeDtypeStruct((N, C), jnp.float32),
    )(p, dis, b2)


def kernel(x, edge_index, W, b):
    row = edge_index[0]
    col = edge_index[1]
    zero64 = jnp.zeros((RPT, C), jnp.float32)
    zero16 = jnp.zeros((RPT, 16), jnp.float32)
    ones16 = jnp.ones((EB, 16), jnp.float32)

    degp = _sc_degree(col, ones16, zero16).reshape(2, N, 16)
    z = _tc_matmul(x, W)
    g0, dis = _tc_prep(degp, z)
    p = _sc_hop(g0, row, col, zero64).reshape(2, N, C)
    g1 = _tc_mid(p, dis)
    q = _sc_hop(g1, row, col, zero64).reshape(2, N, C)
    return _tc_fin(q, dis, b.reshape(1, C))


# speedup vs baseline: 15.0588x; 15.0588x over previous
import functools

import jax
import jax.numpy as jnp
from jax import lax
from jax.experimental import pallas as pl
from jax.experimental.pallas import tpu as pltpu
from jax.experimental.pallas import tpu_sc as plsc

N = 10000
E = 320000
D = 128
C = 64

NC = 2
NS = 16
NW = NC * NS
EPT = E // NW
EB = 80
NBLK = EPT // EB
RPT_A = 632
RPT_B = N - (NS - 1) * RPT_A
FW = C
_cp_untiled = pltpu.CompilerParams(use_tc_tiling_on_sc=False)

_mesh = plsc.VectorSubcoreMesh(core_axis_name="c", subcore_axis_name="s")


@functools.partial(
    pl.kernel,
    out_type=jax.ShapeDtypeStruct((NC * N, 16), jnp.float32),
    mesh=_mesh,
    compiler_params=_cp_untiled,
    scratch_types=[
        pltpu.VMEM((1, EB), jnp.int32),
        pltpu.VMEM((EB, 16), jnp.float32),
        pltpu.VMEM_SHARED((N, 16), jnp.float32),
    ],
)
def _sc_degree(col_hbm, ones_hbm, zero_hbm, out_hbm, idx_c, ones_v, accum):
    cid = lax.axis_index("c")
    sid = lax.axis_index("s")
    wid = cid * NS + sid
    r0 = sid * RPT_A

    def _init(nrows):
        pltpu.sync_copy(zero_hbm.at[pl.ds(0, nrows)],
                        accum.at[pl.ds(r0, nrows)])

    def _drain(nrows):
        pltpu.sync_copy(accum.at[pl.ds(r0, nrows)],
                        out_hbm.at[pl.ds(cid * N + r0, nrows)])

    pl.when(sid < NS - 1)(lambda: _init(RPT_A))
    pl.when(sid == NS - 1)(lambda: _init(RPT_B))
    pltpu.sync_copy(ones_hbm, ones_v)
    plsc.subcore_barrier()
    ebase = wid * EPT

    @pl.loop(0, NBLK)
    def _(j):
        pltpu.sync_copy(col_hbm.at[pl.ds(ebase + j * EB, EB)], idx_c.at[0])
        pltpu.sync_copy(ones_v, accum.at[idx_c.at[0]], add=True)

    plsc.subcore_barrier()
    pl.when(sid < NS - 1)(lambda: _drain(RPT_A))
    pl.when(sid == NS - 1)(lambda: _drain(RPT_B))


@functools.partial(
    pl.kernel,
    out_type=jax.ShapeDtypeStruct((NC * N, FW), jnp.float32),
    mesh=_mesh,
    compiler_params=_cp_untiled,
    scratch_types=[
        pltpu.VMEM((1, EB), jnp.int32),
        pltpu.VMEM((1, EB), jnp.int32),
        pltpu.VMEM((EB, FW), jnp.float32),
        pltpu.VMEM_SHARED((N, FW), jnp.float32),
    ],
)
def _sc_hop(g_hbm, row_hbm, col_hbm, zero_hbm, out_hbm,
            idx_r, idx_c, rows_v, accum):
    cid = lax.axis_index("c")
    sid = lax.axis_index("s")
    wid = cid * NS + sid
    r0 = sid * RPT_A

    def _init(nrows):
        def _f():
            @pl.when(cid == 0)
            def _():
                pltpu.sync_copy(g_hbm.at[pl.ds(r0, nrows)],
                                accum.at[pl.ds(r0, nrows)])

            @pl.when(cid != 0)
            def _():
                pltpu.sync_copy(zero_hbm.at[pl.ds(0, nrows)],
                                accum.at[pl.ds(r0, nrows)])
        return _f

    def _drain(nrows):
        def _f():
            pltpu.sync_copy(accum.at[pl.ds(r0, nrows)],
                            out_hbm.at[pl.ds(cid * N + r0, nrows)])
        return _f

    pl.when(sid < NS - 1)(_init(RPT_A))
    pl.when(sid == NS - 1)(_init(RPT_B))
    plsc.subcore_barrier()
    ebase = wid * EPT

    @pl.loop(0, NBLK)
    def _(j):
        base = ebase + j * EB
        pltpu.sync_copy(row_hbm.at[pl.ds(base, EB)], idx_r.at[0])
        pltpu.sync_copy(col_hbm.at[pl.ds(base, EB)], idx_c.at[0])
        pltpu.sync_copy(g_hbm.at[idx_r.at[0]], rows_v)
        pltpu.sync_copy(rows_v, accum.at[idx_c.at[0]], add=True)

    plsc.subcore_barrier()
    pl.when(sid < NS - 1)(_drain(RPT_A))
    pl.when(sid == NS - 1)(_drain(RPT_B))


RB = 1000


def _z_body(x_ref, w_ref, z_ref):
    z_ref[...] = lax.dot_general(x_ref[...], w_ref[...],
                                 (((1,), (1,)), ((), ())),
                                 preferred_element_type=jnp.float32)


def _tc_matmul(x, W):
    return pl.pallas_call(
        _z_body,
        grid=(N // RB,),
        in_specs=[pl.BlockSpec((RB, D), lambda i: (i, 0)),
                  pl.BlockSpec((C, D), lambda i: (0, 0))],
        out_specs=pl.BlockSpec((RB, C), lambda i: (i, 0)),
        out_shape=jax.ShapeDtypeStruct((N, C), jnp.float32),
    )(x, W)


def _prep_body(degp_ref, z_ref, g0_ref, dis_ref):
    d = degp_ref[...]
    deg = d[0, :, 0] + d[1, :, 0] + 1.0
    dis = lax.rsqrt(deg)
    g0_ref[...] = z_ref[...] * dis[:, None]
    dis_ref[...] = dis[:, None]


def _tc_prep(degp, z):
    return pl.pallas_call(
        _prep_body,
        grid=(N // RB,),
        in_specs=[pl.BlockSpec((2, RB, 16), lambda i: (0, i, 0)),
                  pl.BlockSpec((RB, C), lambda i: (i, 0))],
        out_specs=[pl.BlockSpec((RB, FW), lambda i: (i, 0)),
                   pl.BlockSpec((RB, 1), lambda i: (i, 0))],
        out_shape=[jax.ShapeDtypeStruct((N, FW), jnp.float32),
                   jax.ShapeDtypeStruct((N, 1), jnp.float32)],
    )(degp, z)


def _mid_body(p_ref, dis_ref, o_ref):
    p = p_ref[...]
    dis = dis_ref[...]
    o_ref[...] = (p[0] + p[1]) * (dis * dis)


def _tc_mid(p, dis):
    return pl.pallas_call(
        _mid_body,
        grid=(N // RB,),
        in_specs=[pl.BlockSpec((2, RB, FW), lambda i: (0, i, 0)),
                  pl.BlockSpec((RB, 1), lambda i: (i, 0))],
        out_specs=pl.BlockSpec((RB, FW), lambda i: (i, 0)),
        out_shape=jax.ShapeDtypeStruct((N, FW), jnp.float32),
    )(p, dis)


def _fin_body(p_ref, dis_ref, b_ref, o_ref):
    p = p_ref[...]
    s = (p[0, :, :C] + p[1, :, :C]) * dis_ref[...]
    o_ref[...] = s + b_ref[...]


def _tc_fin(p, dis, b2):
    return pl.pallas_call(
        _fin_body,
        grid=(N // RB,),
        in_specs=[pl.BlockSpec((2, RB, FW), lambda i: (0, i, 0)),
                  pl.BlockSpec((RB, 1), lambda i: (i, 0)),
                  pl.BlockSpec((1, C), lambda i: (0, 0))],
        out_specs=pl.BlockSpec((RB, C), lambda i: (i, 0)),
        out_shape=jax.ShapeDtypeStruct((N, C), jnp.float32),
    )(p, dis, b2)


def kernel(x, edge_index, W, b):
    row = edge_index[0]
    col = edge_index[1]
    zero64 = jnp.zeros((RPT_A, FW), jnp.float32)
    zero16 = jnp.zeros((RPT_A, 16), jnp.float32)
    ones16 = jnp.ones((EB, 16), jnp.float32)

    degp = _sc_degree(col, ones16, zero16).reshape(2, N, 16)
    z = _tc_matmul(x, W)
    g0, dis = _tc_prep(degp, z)
    p = _sc_hop(g0, row, col, zero64).reshape(2, N, FW)
    g1 = _tc_mid(p, dis)
    q = _sc_hop(g1, row, col, zero64).reshape(2, N, FW)
    return _tc_fin(q, dis, b.reshape(1, C))


# speedup vs baseline: 41.6571x; 2.7663x over previous
import functools

import jax
import jax.numpy as jnp
from jax import lax
from jax.experimental import pallas as pl
from jax.experimental.pallas import tpu as pltpu
from jax.experimental.pallas import tpu_sc as plsc

N = 10000
E = 320000
D = 128
C = 64

NC = 2
NS = 16
NW = NC * NS
EPT = E // NW
EB = 125
NBLK = EPT // EB
CH = 8
RPT_A = 632
RPT_B = N - (NS - 1) * RPT_A
FW = C
_cp_untiled = pltpu.CompilerParams(use_tc_tiling_on_sc=False)

_mesh = plsc.VectorSubcoreMesh(core_axis_name="c", subcore_axis_name="s")


@functools.partial(
    pl.kernel,
    out_type=jax.ShapeDtypeStruct((NC * N, 16), jnp.float32),
    mesh=_mesh,
    compiler_params=_cp_untiled,
    scratch_types=[
        pltpu.VMEM((NBLK, EB), jnp.int32),
        pltpu.VMEM((EB, 16), jnp.float32),
        pltpu.VMEM_SHARED((N, 16), jnp.float32),
        pltpu.SemaphoreType.DMA((CH,)),
    ],
)
def _sc_degree(col_hbm, ones_hbm, zero_hbm, out_hbm, idx_all, ones_v, accum,
               sem_s):
    cid = lax.axis_index("c")
    sid = lax.axis_index("s")
    wid = cid * NS + sid
    r0 = sid * RPT_A

    def _init(nrows):
        pltpu.sync_copy(zero_hbm.at[pl.ds(0, nrows)], accum.at[pl.ds(r0, nrows)])

    def _drain(nrows):
        pltpu.sync_copy(accum.at[pl.ds(r0, nrows)],
                        out_hbm.at[pl.ds(cid * N + r0, nrows)])

    pl.when(sid < NS - 1)(lambda: _init(RPT_A))
    pl.when(sid == NS - 1)(lambda: _init(RPT_B))
    pltpu.sync_copy(ones_hbm, ones_v)
    pltpu.sync_copy(col_hbm.at[wid], idx_all)
    plsc.subcore_barrier()

    @pl.loop(0, NBLK, step=CH)
    def _(j0):
        cps = [pltpu.async_copy(ones_v, accum.at[idx_all.at[j0 + i]],
                                sem_s.at[i], add=True) for i in range(CH)]
        for cp in cps:
            cp.wait()

    plsc.subcore_barrier()
    pl.when(sid < NS - 1)(lambda: _drain(RPT_A))
    pl.when(sid == NS - 1)(lambda: _drain(RPT_B))


@functools.partial(
    pl.kernel,
    out_type=jax.ShapeDtypeStruct((NC * N, FW), jnp.float32),
    mesh=_mesh,
    compiler_params=_cp_untiled,
    scratch_types=[
        pltpu.VMEM((NBLK, EB), jnp.int32),
        pltpu.VMEM((NBLK, EB), jnp.int32),
        pltpu.VMEM((CH, EB, FW), jnp.float32),
        pltpu.VMEM_SHARED((N, FW), jnp.float32),
        pltpu.SemaphoreType.DMA((CH,)),
        pltpu.SemaphoreType.DMA((CH,)),
    ],
)
def _sc_hop(g_hbm, row_hbm, col_hbm, zero_hbm, out_hbm,
            idx_r, idx_c, bufs, accum, sem_g, sem_s):
    cid = lax.axis_index("c")
    sid = lax.axis_index("s")
    wid = cid * NS + sid
    r0 = sid * RPT_A

    def _init(nrows):
        def _f():
            @pl.when(cid == 0)
            def _():
                pltpu.sync_copy(g_hbm.at[pl.ds(r0, nrows)],
                                accum.at[pl.ds(r0, nrows)])

            @pl.when(cid != 0)
            def _():
                pltpu.sync_copy(zero_hbm.at[pl.ds(0, nrows)],
                                accum.at[pl.ds(r0, nrows)])
        return _f

    def _drain(nrows):
        def _f():
            pltpu.sync_copy(accum.at[pl.ds(r0, nrows)],
                            out_hbm.at[pl.ds(cid * N + r0, nrows)])
        return _f

    pl.when(sid < NS - 1)(_init(RPT_A))
    pl.when(sid == NS - 1)(_init(RPT_B))
    pltpu.sync_copy(row_hbm.at[wid], idx_r)
    pltpu.sync_copy(col_hbm.at[wid], idx_c)
    plsc.subcore_barrier()

    def _gather(j, i):
        return pltpu.async_copy(g_hbm.at[idx_r.at[j]], bufs.at[i],
                                sem_g.at[i])

    def _scatter(j, i):
        return pltpu.async_copy(bufs.at[i], accum.at[idx_c.at[j]],
                                sem_s.at[i], add=True)

    for i in range(CH):
        _gather(i, i)

    @pl.loop(0, NBLK, step=CH)
    def _(j0):
        for i in range(CH):
            pltpu.make_async_copy(g_hbm.at[idx_r.at[j0 + i]], bufs.at[i],
                                  sem_g.at[i]).wait()
        scs = [_scatter(j0 + i, i) for i in range(CH)]
        for i in range(CH):
            scs[i].wait()

            @pl.when(j0 + CH + i < NBLK)
            def _():
                _gather(j0 + CH + i, i)

    plsc.subcore_barrier()
    pl.when(sid < NS - 1)(_drain(RPT_A))
    pl.when(sid == NS - 1)(_drain(RPT_B))


RB = 1000


def _prep_body(degp_ref, x_ref, w_ref, g0_ref, dis_ref):
    d = degp_ref[...]
    deg = d[0, :, 0] + d[1, :, 0] + 1.0
    dis = lax.rsqrt(deg)
    z = lax.dot_general(x_ref[...], w_ref[...], (((1,), (1,)), ((), ())),
                        preferred_element_type=jnp.float32)
    g0_ref[...] = z * dis[:, None]
    dis_ref[...] = dis[:, None]


def _tc_prep(degp, x, W):
    return pl.pallas_call(
        _prep_body,
        grid=(N // RB,),
        in_specs=[pl.BlockSpec((2, RB, 16), lambda i: (0, i, 0)),
                  pl.BlockSpec((RB, D), lambda i: (i, 0)),
                  pl.BlockSpec((C, D), lambda i: (0, 0))],
        out_specs=[pl.BlockSpec((RB, FW), lambda i: (i, 0)),
                   pl.BlockSpec((RB, 1), lambda i: (i, 0))],
        out_shape=[jax.ShapeDtypeStruct((N, FW), jnp.float32),
                   jax.ShapeDtypeStruct((N, 1), jnp.float32)],
    )(degp, x, W)


def _mid_body(p_ref, dis_ref, o_ref):
    p = p_ref[...]
    dis = dis_ref[...]
    o_ref[...] = (p[0] + p[1]) * (dis * dis)


def _tc_mid(p, dis):
    return pl.pallas_call(
        _mid_body,
        grid=(N // RB,),
        in_specs=[pl.BlockSpec((2, RB, FW), lambda i: (0, i, 0)),
                  pl.BlockSpec((RB, 1), lambda i: (i, 0))],
        out_specs=pl.BlockSpec((RB, FW), lambda i: (i, 0)),
        out_shape=jax.ShapeDtypeStruct((N, FW), jnp.float32),
    )(p, dis)


def _fin_body(p_ref, dis_ref, b_ref, o_ref):
    p = p_ref[...]
    s = (p[0, :, :C] + p[1, :, :C]) * dis_ref[...]
    o_ref[...] = s + b_ref[...]


def _tc_fin(p, dis, b2):
    return pl.pallas_call(
        _fin_body,
        grid=(N // RB,),
        in_specs=[pl.BlockSpec((2, RB, FW), lambda i: (0, i, 0)),
                  pl.BlockSpec((RB, 1), lambda i: (i, 0)),
                  pl.BlockSpec((1, C), lambda i: (0, 0))],
        out_specs=pl.BlockSpec((RB, C), lambda i: (i, 0)),
        out_shape=jax.ShapeDtypeStruct((N, C), jnp.float32),
    )(p, dis, b2)


def kernel(x, edge_index, W, b):
    row = edge_index[0].reshape(NW, NBLK, EB)
    col = edge_index[1].reshape(NW, NBLK, EB)
    zero64 = jnp.zeros((RPT_A, FW), jnp.float32)
    zero16 = jnp.zeros((RPT_A, 16), jnp.float32)
    ones16 = jnp.ones((EB, 16), jnp.float32)

    degp = _sc_degree(col, ones16, zero16).reshape(2, N, 16)
    g0, dis = _tc_prep(degp, x, W)
    p = _sc_hop(g0, row, col, zero64).reshape(2, N, FW)
    g1 = _tc_mid(p, dis)
    q = _sc_hop(g1, row, col, zero64).reshape(2, N, FW)
    return _tc_fin(q, dis, b.reshape(1, C))


# speedup vs baseline: 48.8169x; 1.1719x over previous
import functools

import jax
import jax.numpy as jnp
from jax import lax
from jax.experimental import pallas as pl
from jax.experimental.pallas import tpu as pltpu
from jax.experimental.pallas import tpu_sc as plsc

N = 10000
E = 320000
D = 128
C = 64
CH2 = C // 2

NC = 2
NS = 16
NW = NC * NS
EB = 125
EPT_D = E // NW
NBLK_D = EPT_D // EB
EPT_H = E // NS
NBLK_H = EPT_H // EB
CH = 4
RPT_A = 632
RPT_B = N - (NS - 1) * RPT_A

_mesh = plsc.VectorSubcoreMesh(core_axis_name="c", subcore_axis_name="s")
_cp_untiled = pltpu.CompilerParams(use_tc_tiling_on_sc=False)


@functools.partial(
    pl.kernel,
    out_type=jax.ShapeDtypeStruct((NC * N, 16), jnp.float32),
    mesh=_mesh,
    compiler_params=_cp_untiled,
    scratch_types=[
        pltpu.VMEM((NBLK_D, EB), jnp.int32),
        pltpu.VMEM((EB, 16), jnp.float32),
        pltpu.VMEM_SHARED((N, 16), jnp.float32),
        pltpu.SemaphoreType.DMA((2 * CH,)),
    ],
)
def _sc_degree(col_hbm, ones_hbm, zero_hbm, out_hbm, idx_all, ones_v, accum,
               sem_s):
    cid = lax.axis_index("c")
    sid = lax.axis_index("s")
    wid = cid * NS + sid
    r0 = sid * RPT_A

    def _init(nrows):
        pltpu.sync_copy(zero_hbm.at[pl.ds(0, nrows)], accum.at[pl.ds(r0, nrows)])

    def _drain(nrows):
        pltpu.sync_copy(accum.at[pl.ds(r0, nrows)],
                        out_hbm.at[pl.ds(cid * N + r0, nrows)])

    pl.when(sid < NS - 1)(lambda: _init(RPT_A))
    pl.when(sid == NS - 1)(lambda: _init(RPT_B))
    pltpu.sync_copy(ones_hbm, ones_v)
    pltpu.sync_copy(col_hbm.at[wid], idx_all)
    plsc.subcore_barrier()

    @pl.loop(0, NBLK_D, step=2 * CH)
    def _(j0):
        cps = [pltpu.async_copy(ones_v, accum.at[idx_all.at[j0 + i]],
                                sem_s.at[i], add=True) for i in range(2 * CH)]
        for cp in cps:
            cp.wait()

    plsc.subcore_barrier()
    pl.when(sid < NS - 1)(lambda: _drain(RPT_A))
    pl.when(sid == NS - 1)(lambda: _drain(RPT_B))


@functools.partial(
    pl.kernel,
    out_type=[jax.ShapeDtypeStruct((N, C), jnp.float32),
              jax.ShapeDtypeStruct((N, CH2), jnp.float32),
              jax.ShapeDtypeStruct((N, CH2), jnp.float32)],
    mesh=_mesh,
    compiler_params=_cp_untiled,
    scratch_types=[
        pltpu.VMEM((NBLK_H, EB), jnp.int32),
        pltpu.VMEM((NBLK_H, EB), jnp.int32),
        pltpu.VMEM((2 * CH, EB, CH2), jnp.float32),
        pltpu.VMEM((RPT_A, CH2), jnp.float32),
        pltpu.VMEM((RPT_A, 16), jnp.float32),
        pltpu.VMEM((1, CH2), jnp.float32),
        pltpu.VMEM_SHARED((N, CH2), jnp.float32),
        pltpu.SemaphoreType.DMA((2 * CH,)),
        pltpu.SemaphoreType.DMA((2 * CH,)),
    ],
)
def _sc_megahop(g0a_hbm, g0b_hbm, row_hbm, col_hbm, d2rep_hbm, d1rep_hbm,
                b2_hbm, out_hbm, g1a_hbm, g1b_hbm,
                idx_r, idx_c, bufs, rowstage, repstage, btile, accum,
                sem_g, sem_s):
    cid = lax.axis_index("c")
    sid = lax.axis_index("s")
    r0 = sid * RPT_A

    pltpu.sync_copy(row_hbm.at[sid], idx_r)
    pltpu.sync_copy(col_hbm.at[sid], idx_c)
    pltpu.sync_copy(b2_hbm.at[pl.ds(cid, 1)], btile)

    def _hop(g_hbm):
        def _gather(j, i):
            return pltpu.async_copy(g_hbm.at[idx_r.at[j]], bufs.at[i],
                                    sem_g.at[i])

        def _scatter(j, i):
            return pltpu.async_copy(bufs.at[i], accum.at[idx_c.at[j]],
                                    sem_s.at[i], add=True)

        for i in range(2 * CH):
            _gather(i, i)

        @pl.loop(0, NBLK_H, step=2 * CH)
        def _(j0):
            for half in range(2):
                lo = half * CH
                for i in range(lo, lo + CH):
                    pltpu.make_async_copy(g_hbm.at[idx_r.at[j0 + i]],
                                          bufs.at[i], sem_g.at[i]).wait()
                scs = [_scatter(j0 + i, i) for i in range(lo, lo + CH)]
                for k, i in enumerate(range(lo, lo + CH)):
                    scs[k].wait()

                    @pl.when(j0 + 2 * CH + i < NBLK_H)
                    def _():
                        _gather(j0 + 2 * CH + i, i)

    def _init_accum(g_hbm, nrows):
        pltpu.sync_copy(g_hbm.at[pl.ds(r0, nrows)], accum.at[pl.ds(r0, nrows)])

    def _scale_rows(rep_hbm, nrows, add_bias):
        pltpu.sync_copy(accum.at[pl.ds(r0, nrows)],
                        rowstage.at[pl.ds(0, nrows)])
        pltpu.sync_copy(rep_hbm.at[pl.ds(r0, nrows)],
                        repstage.at[pl.ds(0, nrows)])

        @pl.loop(0, nrows)
        def _(i):
            s = repstage.at[pl.ds(i, 1), pl.ds(0, 16)][...]
            for h in range(CH2 // 16):
                slc = (pl.ds(i, 1), pl.ds(h * 16, 16))
                v = rowstage.at[slc][...] * s
                if add_bias:
                    v = v + btile.at[pl.ds(0, 1), pl.ds(h * 16, 16)][...]
                rowstage.at[slc][...] = v

    def _i1(nrows):
        def _f():
            pl.when(cid == 0)(lambda: _init_accum(g0a_hbm, nrows))
            pl.when(cid != 0)(lambda: _init_accum(g0b_hbm, nrows))
        return _f

    pl.when(sid < NS - 1)(_i1(RPT_A))
    pl.when(sid == NS - 1)(_i1(RPT_B))
    plsc.subcore_barrier()
    pl.when(cid == 0)(lambda: _hop(g0a_hbm))
    pl.when(cid != 0)(lambda: _hop(g0b_hbm))
    plsc.subcore_barrier()

    def _d1(nrows):
        def _f():
            _scale_rows(d2rep_hbm, nrows, add_bias=False)
            pl.when(cid == 0)(lambda: pltpu.sync_copy(
                rowstage.at[pl.ds(0, nrows)], g1a_hbm.at[pl.ds(r0, nrows)]))
            pl.when(cid != 0)(lambda: pltpu.sync_copy(
                rowstage.at[pl.ds(0, nrows)], g1b_hbm.at[pl.ds(r0, nrows)]))
            pltpu.sync_copy(rowstage.at[pl.ds(0, nrows)],
                            accum.at[pl.ds(r0, nrows)])
        return _f

    pl.when(sid < NS - 1)(_d1(RPT_A))
    pl.when(sid == NS - 1)(_d1(RPT_B))
    plsc.subcore_barrier()

    pl.when(cid == 0)(lambda: _hop(g1a_hbm))
    pl.when(cid != 0)(lambda: _hop(g1b_hbm))
    plsc.subcore_barrier()

    def _d2(nrows):
        def _f():
            _scale_rows(d1rep_hbm, nrows, add_bias=True)
            pltpu.sync_copy(
                rowstage.at[pl.ds(0, nrows)],
                out_hbm.at[pl.ds(r0, nrows), pl.ds(cid * CH2, CH2)])
        return _f

    pl.when(sid < NS - 1)(_d2(RPT_A))
    pl.when(sid == NS - 1)(_d2(RPT_B))


RB = 1000


def _prep_body(degp_ref, x_ref, w_ref, g0a_ref, g0b_ref, d2_ref, d1_ref):
    d = degp_ref[...]
    deg = d[0, :, 0] + d[1, :, 0] + 1.0
    y = lax.rsqrt(deg)
    dis = y * (1.5 - 0.5 * deg * y * y)
    z = lax.dot_general(x_ref[...], w_ref[...], (((1,), (1,)), ((), ())),
                        precision=lax.Precision.HIGHEST,
                        preferred_element_type=jnp.float32)
    g0 = z * dis[:, None]
    g0a_ref[...] = g0[:, :CH2]
    g0b_ref[...] = g0[:, CH2:]
    d2_ref[...] = jnp.broadcast_to((dis * dis)[:, None], (RB, 16))
    d1_ref[...] = jnp.broadcast_to(dis[:, None], (RB, 16))


def _tc_prep(degp, x, W):
    return pl.pallas_call(
        _prep_body,
        grid=(N // RB,),
        in_specs=[pl.BlockSpec((2, RB, 16), lambda i: (0, i, 0)),
                  pl.BlockSpec((RB, D), lambda i: (i, 0)),
                  pl.BlockSpec((C, D), lambda i: (0, 0))],
        out_specs=[pl.BlockSpec((RB, CH2), lambda i: (i, 0)),
                   pl.BlockSpec((RB, CH2), lambda i: (i, 0)),
                   pl.BlockSpec((RB, 16), lambda i: (i, 0)),
                   pl.BlockSpec((RB, 16), lambda i: (i, 0))],
        out_shape=[jax.ShapeDtypeStruct((N, CH2), jnp.float32),
                   jax.ShapeDtypeStruct((N, CH2), jnp.float32),
                   jax.ShapeDtypeStruct((N, 16), jnp.float32),
                   jax.ShapeDtypeStruct((N, 16), jnp.float32)],
    )(degp, x, W)


def kernel(x, edge_index, W, b):
    col_d = edge_index[1].reshape(NW, NBLK_D, EB)
    row_h = edge_index[0].reshape(NS, NBLK_H, EB)
    col_h = edge_index[1].reshape(NS, NBLK_H, EB)
    zero16 = jnp.zeros((RPT_A, 16), jnp.float32)
    ones16 = jnp.ones((EB, 16), jnp.float32)

    degp = _sc_degree(col_d, ones16, zero16).reshape(2, N, 16)
    g0a, g0b, d2rep, d1rep = _tc_prep(degp, x, W)
    out, _, _ = _sc_megahop(g0a, g0b, row_h, col_h, d2rep, d1rep,
                            b.reshape(2, CH2))
    return out
